# Initial kernel scaffold; baseline (speedup 1.0000x reference)
import jax
import jax.numpy as jnp
from jax.experimental import pallas as pl


def kernel(x, edge_index, adj_values, W_in, We1, We2, We3, Wz, Wdec, Wg1, Wg2, Wg3, Wg4, Wg5, cluster):
    raise NotImplementedError("write your pallas kernel here")


# speedup vs baseline: 4.0512x; 4.0512x over previous
import functools

import jax
import jax.numpy as jnp
from jax import lax
from jax.experimental import pallas as pl
from jax.experimental.pallas import tpu as pltpu
from jax.experimental.pallas import tpu_sc as plsc

N = 10000
E = 320000
D = 128
NC = 4
D5 = 128
SIGMA = 0.5

NCORE = 2
NSUB = 16
NW = NCORE * NSUB
CHUNK = 128

E_PAD = ((E + NW * CHUNK - 1) // (NW * CHUNK)) * (NW * CHUNK)
EPW = E_PAD // NW
NCHUNK = EPW // CHUNK
RPS = 624

BR = 1000


def _make_spmm(d):
  mesh = plsc.VectorSubcoreMesh(core_axis_name="c", subcore_axis_name="s",
                                num_cores=NCORE, num_subcores=NSUB)

  @functools.partial(
      pl.kernel,
      out_type=jax.ShapeDtypeStruct((NCORE, N, d), jnp.float32),
      mesh=mesh,
      scratch_types=[
          pltpu.VMEM((EPW,), jnp.int32),
          pltpu.VMEM((EPW,), jnp.float32),
          pltpu.VMEM((CHUNK,), jnp.int32),
          pltpu.VMEM((CHUNK, d), jnp.float32),
          pltpu.VMEM_SHARED((N, d), jnp.float32),
          pltpu.SemaphoreType.DMA,
      ],
  )
  def spmm(sup_hbm, src_hbm, dst_hbm, adj_hbm, out_hbm,
           src_v, adj_v, dst_v, rows_v, acc, sem):
    cid = lax.axis_index("c")
    sid = lax.axis_index("s")
    wid = sid * NCORE + cid
    base = wid * EPW

    def zrow(i, _):
      for t in range(d // 16):
        rows_v[i, pl.ds(t * 16, 16)] = jnp.zeros((16,), jnp.float32)
      return 0
    lax.fori_loop(0, CHUNK, zrow, 0)
    full, rem = RPS // CHUNK, RPS % CHUNK
    for j in range(full):
      pltpu.sync_copy(rows_v, acc.at[pl.ds(sid * RPS + j * CHUNK, CHUNK)])
    if rem:
      pltpu.sync_copy(rows_v.at[pl.ds(0, rem)],
                      acc.at[pl.ds(sid * RPS + full * CHUNK, rem)])
    tail = N - NSUB * RPS
    @pl.when(sid == NSUB - 1)
    def _():
      pltpu.sync_copy(rows_v.at[pl.ds(0, tail)],
                      acc.at[pl.ds(NSUB * RPS, tail)])

    pltpu.sync_copy(src_hbm.at[pl.ds(base, EPW)], src_v)
    pltpu.sync_copy(adj_hbm.at[pl.ds(base, EPW)], adj_v)
    plsc.subcore_barrier()

    def chunk_body(k, _):
      koff = k * CHUNK
      pltpu.sync_copy(dst_hbm.at[pl.ds(base + koff, CHUNK)], dst_v)
      pltpu.async_copy(sup_hbm.at[src_v.at[pl.ds(koff, CHUNK)]],
                       rows_v, sem).wait()
      def sgrp(g, _):
        a16 = adj_v[pl.ds(koff + g * 16, 16)]
        for j in range(16):
          a = a16[j]
          r = g * 16 + j
          for t in range(d // 16):
            sl = pl.ds(t * 16, 16)
            rows_v[r, sl] = rows_v[r, sl] * a
        return 0
      lax.fori_loop(0, CHUNK // 16, sgrp, 0)
      pltpu.sync_copy(rows_v, acc.at[dst_v], add=True)
      return 0
    lax.fori_loop(0, NCHUNK, chunk_body, 0)

    plsc.subcore_barrier()
    pltpu.sync_copy(acc.at[pl.ds(sid * RPS, RPS)],
                    out_hbm.at[cid, pl.ds(sid * RPS, RPS)])
    @pl.when(sid == NSUB - 1)
    def _():
      pltpu.sync_copy(acc.at[pl.ds(NSUB * RPS, tail)],
                      out_hbm.at[cid, pl.ds(NSUB * RPS, tail)])

  return spmm


_spmm128 = _make_spmm(D)


def _enc_body(flat, win, we1, we2, we3, wz, wdec, wg1, clus,
              tra1, tra2, tra3, zo, xbar, s1, q):
  f = flat[...]
  enc = jnp.maximum(jnp.dot(f, win[...], preferred_element_type=jnp.float32), 0.0)
  t1 = jnp.maximum(jnp.dot(enc, we1[...], preferred_element_type=jnp.float32), 0.0)
  t2 = jnp.maximum(jnp.dot(t1, we2[...], preferred_element_type=jnp.float32), 0.0)
  t3 = jnp.maximum(jnp.dot(t2, we3[...], preferred_element_type=jnp.float32), 0.0)
  z = jnp.dot(t3, wz[...], preferred_element_type=jnp.float32)
  xbar[...] = jnp.dot(z, wdec[...], preferred_element_type=jnp.float32)
  tra1[...] = t1
  tra2[...] = t2
  tra3[...] = t3
  zo[...] = z
  s1[...] = jnp.dot(enc, wg1[...], preferred_element_type=jnp.float32)
  c = clus[...]
  zc = lax.dot_general(z, c, (((1,), (1,)), ((), ())),
                       preferred_element_type=jnp.float32)
  z2 = jnp.sum(z * z, axis=1, keepdims=True)
  c2 = jnp.sum(c * c, axis=1)[None, :]
  qv = 1.0 / (1.0 + (z2 - 2.0 * zc + c2))
  q[...] = qv / jnp.sum(qv, axis=1, keepdims=True)


def _encoder(flat, win, we1, we2, we3, wz, wdec, wg1, clus):
  g = N // BR
  row = pl.BlockSpec((BR, D), lambda i: (i, 0))
  full = pl.BlockSpec((D, D), lambda i: (0, 0))
  outs = [jax.ShapeDtypeStruct((N, D), jnp.float32)] * 6 + [
      jax.ShapeDtypeStruct((N, NC), jnp.float32)]
  return pl.pallas_call(
      _enc_body,
      grid=(g,),
      in_specs=[row, full, full, full, full, full, full, full,
                pl.BlockSpec((NC, D), lambda i: (0, 0))],
      out_specs=[row] * 6 + [pl.BlockSpec((BR, NC), lambda i: (i, 0))],
      out_shape=outs,
  )(flat, win, we1, we2, we3, wz, wdec, wg1, clus)


def _layer_body(p0, p1, tra, wg, out):
  h = jnp.maximum(p0[...] + p1[...], 0.0)
  s = (1.0 - SIGMA) * h + SIGMA * tra[...]
  out[...] = jnp.dot(s, wg[...], preferred_element_type=jnp.float32)


def _layer(p0, p1, tra, wg, dout):
  g = N // BR
  row = pl.BlockSpec((BR, D), lambda i: (i, 0))
  return pl.pallas_call(
      _layer_body,
      grid=(g,),
      in_specs=[row, row, row, pl.BlockSpec((D, dout), lambda i: (0, 0))],
      out_specs=pl.BlockSpec((BR, dout), lambda i: (i, 0)),
      out_shape=jax.ShapeDtypeStruct((N, dout), jnp.float32),
  )(p0, p1, tra, wg)


def _final_body(p0, p1, out):
  h = p0[...] + p1[...]
  hv = h[:, :NC]
  m = jnp.max(hv, axis=1, keepdims=True)
  e = jnp.exp(hv - m)
  out[...] = e / jnp.sum(e, axis=1, keepdims=True)


def _final(p0, p1):
  g = N // BR
  row = pl.BlockSpec((BR, D5), lambda i: (i, 0))
  return pl.pallas_call(
      _final_body,
      grid=(g,),
      in_specs=[row, row],
      out_specs=pl.BlockSpec((BR, NC), lambda i: (i, 0)),
      out_shape=jax.ShapeDtypeStruct((N, NC), jnp.float32),
  )(p0, p1)


def kernel(x, edge_index, adj_values, W_in, We1, We2, We3, Wz, Wdec,
           Wg1, Wg2, Wg3, Wg4, Wg5, cluster):
  flat = x.reshape(N, -1)

  pad = E_PAD - E
  src = jnp.pad(edge_index[1], (0, pad))
  dst = jnp.pad(edge_index[0], (0, pad))
  adj = jnp.pad(adj_values, (0, pad))

  tra1, tra2, tra3, z, xbar, s1, q = _encoder(
      flat, W_in, We1, We2, We3, Wz, Wdec, Wg1, cluster)

  p = _spmm128(s1, src, dst, adj)
  s2 = _layer(p[0], p[1], tra1, Wg2, D)
  p = _spmm128(s2, src, dst, adj)
  s3 = _layer(p[0], p[1], tra2, Wg3, D)
  p = _spmm128(s3, src, dst, adj)
  s4 = _layer(p[0], p[1], tra3, Wg4, D)
  p = _spmm128(s4, src, dst, adj)
  wg5p = jnp.pad(Wg5, ((0, 0), (0, D5 - NC)))
  s5 = _layer(p[0], p[1], z, wg5p, D5)
  p = _spmm128(s5, src, dst, adj)
  predict = _final(p[0], p[1])

  return (xbar.reshape(x.shape), q, predict)
